# baseline (speedup 1.0000x reference)
import functools

import jax
import jax.numpy as jnp
from jax import lax
from jax.experimental import pallas as pl
from jax.experimental.pallas import tpu as pltpu
from jax.experimental.pallas import tpu_sc as plsc

N_ROWS = 8192
N_CODES = 8192
DIM = 256
LOSS_SCALE = 1.25 / (N_ROWS * DIM)


def _transpose_body(e_ref, o_ref):
    o_ref[...] = e_ref[...].T


def _transpose_e(embeddings):
    bt = 8192
    return pl.pallas_call(
        _transpose_body,
        grid=(N_CODES // bt,),
        in_specs=[pl.BlockSpec((DIM, bt), lambda i: (0, i))],
        out_specs=pl.BlockSpec((bt, DIM), lambda i: (i, 0)),
        out_shape=jax.ShapeDtypeStruct((N_CODES, DIM), jnp.float32),
    )(embeddings)


_LB = 8192


def _loss_body(q_ref, f_ref, loss_ref):
    r = pl.program_id(0)
    d = q_ref[...] - f_ref[...]
    bsum = jnp.sum(d * d) * LOSS_SCALE

    @pl.when(r == 0)
    def _():
        loss_ref[0, 0] = bsum

    @pl.when(r > 0)
    def _():
        loss_ref[0, 0] += bsum


def _loss(q, flat):
    return pl.pallas_call(
        _loss_body,
        grid=(N_ROWS // _LB,),
        in_specs=[
            pl.BlockSpec((_LB, DIM), lambda r: (r, 0)),
            pl.BlockSpec((_LB, DIM), lambda r: (r, 0)),
        ],
        out_specs=pl.BlockSpec(memory_space=pltpu.SMEM, block_shape=(1, 1),
                               index_map=lambda r: (0, 0)),
        out_shape=jax.ShapeDtypeStruct((1, 1), jnp.float32),
    )(q, flat)


_SC_INFO = plsc.get_sparse_core_info()
_NW = _SC_INFO.num_cores * _SC_INFO.num_subcores
_B_PER_W = N_ROWS // _NW


@functools.partial(
    pl.kernel,
    out_type=jax.ShapeDtypeStruct((N_ROWS, DIM), jnp.float32),
    mesh=plsc.VectorSubcoreMesh(core_axis_name="c", subcore_axis_name="s"),
    scratch_types=[
        pltpu.VMEM((_B_PER_W,), jnp.int32),
        pltpu.VMEM((_B_PER_W, DIM), jnp.float32),
        pltpu.SemaphoreType.DMA,
    ],
)
def _sc_gather(idx_hbm, table_hbm, out_hbm, idx_v, rows_v, sem):
    wid = lax.axis_index("s") * _SC_INFO.num_cores + lax.axis_index("c")
    base = wid * _B_PER_W
    pltpu.sync_copy(idx_hbm.at[pl.ds(base, _B_PER_W)], idx_v)
    pltpu.async_copy(table_hbm.at[idx_v], rows_v, sem).wait()
    pltpu.sync_copy(rows_v, out_hbm.at[pl.ds(base, _B_PER_W)])


def kernel(inputs, embeddings):
    flat = inputs.reshape(N_ROWS, DIM)
    distances = (
        jnp.sum(flat ** 2, axis=1, keepdims=True)
        - 2.0 * (flat @ embeddings)
        + jnp.sum(embeddings ** 2, axis=0, keepdims=True)
    )
    encoding_indices = jnp.argmin(distances, axis=1)
    et = _transpose_e(embeddings)
    quantized = _sc_gather(encoding_indices, et)
    loss = _loss(quantized, flat)[0, 0]
    return (
        quantized.reshape(inputs.shape),
        encoding_indices.reshape(inputs.shape[:-1]),
        loss,
    )


# speedup vs baseline: 1.0187x; 1.0187x over previous
import functools

import jax
import jax.numpy as jnp
from jax import lax
from jax.experimental import pallas as pl
from jax.experimental.pallas import tpu as pltpu
from jax.experimental.pallas import tpu_sc as plsc

N_ROWS = 8192
N_CODES = 8192
DIM = 256
LOSS_SCALE = 1.25 / (N_ROWS * DIM)


def _transpose_body(e_ref, o_ref):
    o_ref[...] = e_ref[...].T


def _transpose_e(embeddings):
    bt = 4096
    return pl.pallas_call(
        _transpose_body,
        grid=(N_CODES // bt,),
        in_specs=[pl.BlockSpec((DIM, bt), lambda i: (0, i))],
        out_specs=pl.BlockSpec((bt, DIM), lambda i: (i, 0)),
        out_shape=jax.ShapeDtypeStruct((N_CODES, DIM), jnp.float32),
    )(embeddings)


_LB = 4096


def _loss_body(q_ref, f_ref, loss_ref):
    r = pl.program_id(0)
    d = q_ref[...] - f_ref[...]
    bsum = jnp.sum(d * d) * LOSS_SCALE

    @pl.when(r == 0)
    def _():
        loss_ref[0, 0] = bsum

    @pl.when(r > 0)
    def _():
        loss_ref[0, 0] += bsum


def _loss(q, flat):
    return pl.pallas_call(
        _loss_body,
        grid=(N_ROWS // _LB,),
        in_specs=[
            pl.BlockSpec((_LB, DIM), lambda r: (r, 0)),
            pl.BlockSpec((_LB, DIM), lambda r: (r, 0)),
        ],
        out_specs=pl.BlockSpec(memory_space=pltpu.SMEM, block_shape=(1, 1),
                               index_map=lambda r: (0, 0)),
        out_shape=jax.ShapeDtypeStruct((1, 1), jnp.float32),
    )(q, flat)


_SC_INFO = plsc.get_sparse_core_info()
_NW = _SC_INFO.num_cores * _SC_INFO.num_subcores
_B_PER_W = N_ROWS // _NW


@functools.partial(
    pl.kernel,
    out_type=jax.ShapeDtypeStruct((N_ROWS, DIM), jnp.float32),
    mesh=plsc.VectorSubcoreMesh(core_axis_name="c", subcore_axis_name="s"),
    scratch_types=[
        pltpu.VMEM((_B_PER_W,), jnp.int32),
        pltpu.VMEM((_B_PER_W, DIM), jnp.float32),
        pltpu.SemaphoreType.DMA,
    ],
)
def _sc_gather(idx_hbm, table_hbm, out_hbm, idx_v, rows_v, sem):
    wid = lax.axis_index("s") * _SC_INFO.num_cores + lax.axis_index("c")
    base = wid * _B_PER_W
    pltpu.sync_copy(idx_hbm.at[pl.ds(base, _B_PER_W)], idx_v)
    pltpu.async_copy(table_hbm.at[idx_v], rows_v, sem).wait()
    pltpu.sync_copy(rows_v, out_hbm.at[pl.ds(base, _B_PER_W)])


def kernel(inputs, embeddings):
    flat = inputs.reshape(N_ROWS, DIM)
    distances = (
        jnp.sum(flat ** 2, axis=1, keepdims=True)
        - 2.0 * (flat @ embeddings)
        + jnp.sum(embeddings ** 2, axis=0, keepdims=True)
    )
    encoding_indices = jnp.argmin(distances, axis=1)
    et = _transpose_e(embeddings)
    quantized = _sc_gather(encoding_indices, et)
    loss = _loss(quantized, flat)[0, 0]
    return (
        quantized.reshape(inputs.shape),
        encoding_indices.reshape(inputs.shape[:-1]),
        loss,
    )
